# baseline (speedup 1.0000x reference)
import functools

import jax
import jax.numpy as jnp
from jax import lax
from jax.experimental import pallas as pl
from jax.experimental.pallas import tpu as pltpu
from jax.experimental.pallas import tpu_sc as plsc

HID = 768
EPS = 1e-6
L = 16
NV = HID // L
NC = 2
NS = 16
NW = NC * NS
CHUNK = 32
RGRP = 4


def _lanesum(x):
    lane = lax.iota(jnp.int32, L)
    for m in (1, 2, 4, 8):
        x = x + x.at[lane ^ m].get(mode="promise_in_bounds")
    return x


def _rsqrt16(v):
    i = lax.bitcast_convert_type(v, jnp.int32)
    y = lax.bitcast_convert_type(jnp.int32(0x5F3759DF) - (i >> 1), jnp.float32)
    h = v * 0.5
    for _ in range(3):
        y = y * (1.5 - h * y * y)
    return y


@functools.cache
def _build(n_tokens, seq):
    rows_per_w = n_tokens // NW
    nchunks = rows_per_w // CHUNK
    assert nchunks >= 2
    mesh = plsc.VectorSubcoreMesh(core_axis_name="c", subcore_axis_name="s")

    @functools.partial(
        pl.kernel,
        mesh=mesh,
        out_type=jax.ShapeDtypeStruct((n_tokens, HID), jnp.float32),
        scratch_types=[
            pltpu.VMEM((rows_per_w,), jnp.int32),
            pltpu.VMEM((2, CHUNK, HID), jnp.float32),
            pltpu.VMEM((2, CHUNK, HID), jnp.float32),
            pltpu.VMEM((HID,), jnp.float32),
            pltpu.VMEM((HID,), jnp.float32),
            pltpu.SemaphoreType.DMA((2,)),
            pltpu.SemaphoreType.DMA((2,)),
            pltpu.SemaphoreType.DMA((2,)),
        ],
    )
    def k(ids_hbm, word_hbm, pos_hbm, gamma_hbm, beta_hbm, out_hbm,
          idx_v, wbuf, pbuf, gv, bv, sem_w, sem_p, sem_o):
        wid = lax.axis_index("s") * NC + lax.axis_index("c")
        base = wid * rows_per_w
        s0 = base % seq

        pltpu.sync_copy(ids_hbm.at[pl.ds(base, rows_per_w)], idx_v)
        pltpu.sync_copy(gamma_hbm, gv)
        pltpu.sync_copy(beta_hbm, bv)

        def in_copies(c, par):
            row0 = pl.multiple_of(c * CHUNK, CHUNK)
            return (
                pltpu.make_async_copy(pos_hbm.at[pl.ds(s0 + row0, CHUNK)],
                                      pbuf.at[par], sem_p.at[par]),
                pltpu.make_async_copy(word_hbm.at[idx_v.at[pl.ds(row0, CHUNK)]],
                                      wbuf.at[par], sem_w.at[par]),
            )

        def out_copy(c, par):
            row0 = pl.multiple_of(c * CHUNK, CHUNK)
            return pltpu.make_async_copy(
                wbuf.at[par], out_hbm.at[pl.ds(base + row0, CHUNK)],
                sem_o.at[par])

        def chunk_body(c, carry):
            par = jnp.bitwise_and(c, 1)
            npar = 1 - par
            nxt = c + 1

            @pl.when(nxt < nchunks)
            def _prefetch():
                @pl.when(nxt >= 2)
                def _drain_out():
                    out_copy(nxt - 2, npar).wait()
                for cp in in_copies(nxt, npar):
                    cp.start()

            for cp in in_copies(c, par):
                cp.wait()

            def grp_body(g, gc):
                r0 = g * RGRP
                means, istds = [], []
                for i in range(RGRP):
                    r = r0 + i
                    vsum = jnp.zeros((L,), jnp.float32)
                    vsq = jnp.zeros((L,), jnp.float32)
                    for j in range(NV):
                        sl = pl.ds(j * L, L)
                        x = wbuf[par, r, sl] + pbuf[par, r, sl]
                        wbuf[par, r, sl] = x
                        vsum = vsum + x
                        vsq = vsq + x * x
                    mean_v = _lanesum(vsum) * (1.0 / HID)
                    var_v = _lanesum(vsq) * (1.0 / HID) - mean_v * mean_v
                    means.append(mean_v)
                    istds.append(_rsqrt16(var_v + EPS))
                for j in range(NV):
                    sl = pl.ds(j * L, L)
                    gj = gv[sl]
                    bj = bv[sl]
                    for i in range(RGRP):
                        r = r0 + i
                        x = wbuf[par, r, sl]
                        wbuf[par, r, sl] = (x - means[i]) * istds[i] * gj + bj
                return gc

            lax.fori_loop(0, CHUNK // RGRP, grp_body, 0)
            out_copy(c, par).start()
            return carry

        in_copies(0, 0)[0].start()
        in_copies(0, 0)[1].start()
        lax.fori_loop(0, nchunks, chunk_body, 0)
        out_copy(nchunks - 2, (nchunks - 2) % 2).wait()
        out_copy(nchunks - 1, (nchunks - 1) % 2).wait()

    return k


def kernel(input_ids, word_embeddings, position_embeddings, gamma, beta):
    b, s = input_ids.shape
    ids = input_ids.reshape(-1).astype(jnp.int32)
    out = _build(b * s, s)(ids, word_embeddings, position_embeddings, gamma, beta)
    return out.reshape(b, s, HID)
